# baseline (speedup 1.0000x reference)
import jax
import jax.numpy as jnp
from jax import lax
from jax.experimental import pallas as pl
from jax.experimental.pallas import tpu as pltpu
from jax.experimental.pallas import tpu_sc as plsc

N = 10000
NPAD = 10240
E = 320000
F = 128
K = 64

NC = 2
NS = 16
NW = NC * NS
EPW = E // NW
CH = 80
NCHUNK = EPW // CH
NB = 5
NT = NCHUNK // NB
CHS = 40
NCHUNKS = EPW // CHS
NTS = NCHUNKS // NB
RPT = NPAD // NS
DEGW = 8

_f32 = jnp.float32
_SC_PARAMS = dict(use_tc_tiling_on_sc=False, needs_layout_passes=False)


def _mesh():
  return plsc.VectorSubcoreMesh(core_axis_name="c", subcore_axis_name="s")


def _ids():
  c = lax.axis_index("c")
  s = lax.axis_index("s")
  return c, s, c * NS + s


def _sc_prep_body(dst3, row3, col3, a_hbm, b_hbm, ones_hbm, zer_hbm,
                  deg_hbm, efa_hbm, efb_hbm,
                  idd_v, ira_v, icb_v, ra_v, rb_v, ones_v, deg_s, *sems):
  dsem = sems[0:NB]
  ga = sems[NB:2 * NB]
  sa = sems[2 * NB:3 * NB]
  gb = sems[3 * NB:4 * NB]
  sb = sems[4 * NB:5 * NB]
  c, s, w = _ids()
  base = w * EPW
  pltpu.sync_copy(dst3.at[w], idd_v)
  pltpu.sync_copy(row3.at[w], ira_v)
  pltpu.sync_copy(col3.at[w], icb_v)
  pltpu.sync_copy(ones_hbm, ones_v)
  pltpu.sync_copy(zer_hbm, deg_s.at[pl.ds(s * RPT, RPT)])
  for b in range(NB):
    pltpu.async_copy(a_hbm.at[ira_v.at[b]], ra_v.at[b], ga[b])
    pltpu.async_copy(b_hbm.at[icb_v.at[b]], rb_v.at[b], gb[b])
  plsc.subcore_barrier()

  def rnd(t, carry):
    for b in range(NB):
      g = t * NB + b
      e0 = base + g * CH

      @pl.when(t > 0)
      def _wait_deg():
        pltpu.make_async_copy(ones_v, deg_s.at[idd_v.at[g - NB]],
                              dsem[b]).wait()

      pltpu.async_copy(ones_v, deg_s.at[idd_v.at[g]], dsem[b], add=True)

      pltpu.make_async_copy(a_hbm.at[ira_v.at[g]], ra_v.at[b], ga[b]).wait()
      pltpu.async_copy(ra_v.at[b], efa_hbm.at[pl.ds(e0, CH)], sa[b])
      pltpu.make_async_copy(b_hbm.at[icb_v.at[g]], rb_v.at[b], gb[b]).wait()
      pltpu.async_copy(rb_v.at[b], efb_hbm.at[pl.ds(e0, CH)], sb[b])

    @pl.when(t < NT - 1)
    def _next():
      for b in range(NB):
        g = t * NB + b
        g2 = g + NB
        e0 = base + g * CH
        pltpu.make_async_copy(ra_v.at[b], efa_hbm.at[pl.ds(e0, CH)],
                              sa[b]).wait()
        pltpu.async_copy(a_hbm.at[ira_v.at[g2]], ra_v.at[b], ga[b])
        pltpu.make_async_copy(rb_v.at[b], efb_hbm.at[pl.ds(e0, CH)],
                              sb[b]).wait()
        pltpu.async_copy(b_hbm.at[icb_v.at[g2]], rb_v.at[b], gb[b])

    return carry

  lax.fori_loop(0, NT, rnd, 0)
  for b in range(NB):
    g = (NT - 1) * NB + b
    e0 = base + g * CH
    pltpu.make_async_copy(ones_v, deg_s.at[idd_v.at[g]], dsem[b]).wait()
    pltpu.make_async_copy(ra_v.at[b], efa_hbm.at[pl.ds(e0, CH)], sa[b]).wait()
    pltpu.make_async_copy(rb_v.at[b], efb_hbm.at[pl.ds(e0, CH)], sb[b]).wait()
  plsc.subcore_barrier()
  pltpu.sync_copy(deg_s.at[pl.ds(s * RPT, RPT)],
                  deg_hbm.at[c, pl.ds(s * RPT, RPT)])


def _sc_prep(dst3, row3, col3, atab, btab):
  fn = pl.kernel(
      _sc_prep_body,
      out_type=(
          jax.ShapeDtypeStruct((NC, NPAD, DEGW), _f32),
          jax.ShapeDtypeStruct((E, 32), _f32),
          jax.ShapeDtypeStruct((E, 32), _f32),
      ),
      mesh=_mesh(),
      compiler_params=pltpu.CompilerParams(**_SC_PARAMS),
      scratch_types=[
          pltpu.VMEM((NCHUNK, CH), jnp.int32),
          pltpu.VMEM((NCHUNK, CH), jnp.int32),
          pltpu.VMEM((NCHUNK, CH), jnp.int32),
          pltpu.VMEM((NB, CH, 32), _f32),
          pltpu.VMEM((NB, CH, 32), _f32),
          pltpu.VMEM((CH, DEGW), _f32),
          pltpu.VMEM_SHARED((NPAD, DEGW), _f32),
      ] + [pltpu.SemaphoreType.DMA] * (5 * NB),
  )
  ones = jnp.ones((CH, DEGW), _f32)
  zer = jnp.zeros((RPT, DEGW), _f32)
  return fn(dst3, row3, col3, atab, btab, ones, zer)


def _sc_seg_body(src3, dst3, y_hbm, zer_hbm, acc_hbm,
                 isrc_v, idst_v, rows_v, acc_s, *sems, ch, nt):
  gy = sems[0:NB]
  ss = sems[NB:2 * NB]
  c, s, w = _ids()
  pltpu.sync_copy(src3.at[w], isrc_v)
  pltpu.sync_copy(dst3.at[w], idst_v)
  pltpu.sync_copy(zer_hbm, acc_s.at[pl.ds(s * RPT, RPT)])
  for b in range(NB):
    pltpu.async_copy(y_hbm.at[isrc_v.at[b]], rows_v.at[b], gy[b])
  plsc.subcore_barrier()

  def rnd(t, carry):
    for b in range(NB):
      g = t * NB + b
      pltpu.make_async_copy(y_hbm.at[isrc_v.at[g]], rows_v.at[b], gy[b]).wait()
      pltpu.async_copy(rows_v.at[b], acc_s.at[idst_v.at[g]], ss[b], add=True)

    @pl.when(t < nt - 1)
    def _next():
      for b in range(NB):
        g = t * NB + b
        pltpu.make_async_copy(rows_v.at[b], acc_s.at[idst_v.at[g]],
                              ss[b]).wait()
        pltpu.async_copy(y_hbm.at[isrc_v.at[g + NB]], rows_v.at[b], gy[b])

    return carry

  lax.fori_loop(0, nt, rnd, 0)
  for b in range(NB):
    g = (nt - 1) * NB + b
    pltpu.make_async_copy(rows_v.at[b], acc_s.at[idst_v.at[g]], ss[b]).wait()
  plsc.subcore_barrier()
  pltpu.sync_copy(acc_s.at[pl.ds(s * RPT, RPT)],
                  acc_hbm.at[c, pl.ds(s * RPT, RPT)])


def _sc_segsum(src, dst, y, d, ch):
  import functools as _ft
  nchunk = EPW // ch
  nt = nchunk // NB
  body = _ft.partial(_sc_seg_body, ch=ch, nt=nt)
  fn = pl.kernel(
      body,
      out_type=jax.ShapeDtypeStruct((NC, NPAD, d), _f32),
      mesh=_mesh(),
      compiler_params=pltpu.CompilerParams(**_SC_PARAMS),
      scratch_types=[
          pltpu.VMEM((nchunk, ch), jnp.int32),
          pltpu.VMEM((nchunk, ch), jnp.int32),
          pltpu.VMEM((NB, ch, d), _f32),
          pltpu.VMEM_SHARED((NPAD, d), _f32),
      ] + [pltpu.SemaphoreType.DMA] * (2 * NB),
  )
  return fn(src.reshape(NW, nchunk, ch), dst.reshape(NW, nchunk, ch), y,
            jnp.zeros((RPT, d), _f32))


def _sc_wseg_body(row3, col3, w3, z_hbm, zer_hbm, acc_hbm,
                  irow_v, icol_v, wv_v, rows_v, acc_s, *sems):
  gz = sems[0:NB]
  ss = sems[NB:2 * NB]
  c, s, w = _ids()
  pltpu.sync_copy(row3.at[w], irow_v)
  pltpu.sync_copy(col3.at[w], icol_v)
  pltpu.sync_copy(w3.at[w], wv_v)
  pltpu.sync_copy(zer_hbm, acc_s.at[pl.ds(s * RPT, RPT)])
  for b in range(NB):
    pltpu.async_copy(z_hbm.at[icol_v.at[b]], rows_v.at[b], gz[b])
  plsc.subcore_barrier()

  def rnd(t, carry):
    for b in range(NB):
      g = t * NB + b
      pltpu.make_async_copy(z_hbm.at[icol_v.at[g]], rows_v.at[b], gz[b]).wait()

      def egrp(eb, ecarry):
        e0 = eb * 16
        for u in range(16):
          e = e0 + u
          wb = plsc.load_gather(
              wv_v,
              [jnp.full((16,), g, jnp.int32), jnp.full((16,), e, jnp.int32)])
          for j in range(K // 16):
            rows_v[b, e, pl.ds(j * 16, 16)] = (
                rows_v[b, e, pl.ds(j * 16, 16)] * wb)
        return ecarry

      lax.fori_loop(0, CH // 16, egrp, 0)
      pltpu.async_copy(rows_v.at[b], acc_s.at[irow_v.at[g]], ss[b], add=True)

    @pl.when(t < NT - 1)
    def _next():
      for b in range(NB):
        g = t * NB + b
        pltpu.make_async_copy(rows_v.at[b], acc_s.at[irow_v.at[g]],
                              ss[b]).wait()
        pltpu.async_copy(z_hbm.at[icol_v.at[g + NB]], rows_v.at[b], gz[b])

    return carry

  lax.fori_loop(0, NT, rnd, 0)
  for b in range(NB):
    g = (NT - 1) * NB + b
    pltpu.make_async_copy(rows_v.at[b], acc_s.at[irow_v.at[g]], ss[b]).wait()
  plsc.subcore_barrier()
  pltpu.sync_copy(acc_s.at[pl.ds(s * RPT, RPT)],
                  acc_hbm.at[c, pl.ds(s * RPT, RPT)])


def _sc_wseg(row3, col3, w3, z):
  fn = pl.kernel(
      _sc_wseg_body,
      out_type=jax.ShapeDtypeStruct((NC, NPAD, K), _f32),
      mesh=_mesh(),
      compiler_params=pltpu.CompilerParams(**_SC_PARAMS),
      scratch_types=[
          pltpu.VMEM((NCHUNK, CH), jnp.int32),
          pltpu.VMEM((NCHUNK, CH), jnp.int32),
          pltpu.VMEM((NCHUNK, CH), _f32),
          pltpu.VMEM((NB, CH, K), _f32),
          pltpu.VMEM_SHARED((NPAD, K), _f32),
      ] + [pltpu.SemaphoreType.DMA] * (2 * NB),
  )
  return fn(row3, col3, w3, z, jnp.zeros((RPT, K), _f32))


BM = 1024
NBM = NPAD // BM


def _mm_body(x_ref, w_ref, o_ref):
  o_ref[...] = jnp.dot(x_ref[...], w_ref[...],
                       preferred_element_type=_f32)


def _tc_mm(x, w):
  kn = w.shape[1]
  return pl.pallas_call(
      _mm_body,
      grid=(NBM,),
      in_specs=[
          pl.BlockSpec((BM, F), lambda i: (i, 0)),
          pl.BlockSpec((F, kn), lambda i: (0, 0)),
      ],
      out_specs=pl.BlockSpec((BM, kn), lambda i: (i, 0)),
      out_shape=jax.ShapeDtypeStruct((NPAD, kn), _f32),
  )(x, w)


def _tc2_body(degp_ref, xw_ref, y1_ref, dinv_ref):
  deg = degp_ref[0] + degp_ref[1]
  dinv = lax.rsqrt(deg[:, 0:1] + 1.0)
  dinv_ref[...] = dinv
  y1_ref[...] = xw_ref[...] * dinv


def _tc2(degp, xw1):
  return pl.pallas_call(
      _tc2_body,
      grid=(NBM,),
      in_specs=[
          pl.BlockSpec((NC, BM, DEGW), lambda i: (0, i, 0)),
          pl.BlockSpec((BM, F), lambda i: (i, 0)),
      ],
      out_specs=(
          pl.BlockSpec((BM, F), lambda i: (i, 0)),
          pl.BlockSpec((BM, 1), lambda i: (i, 0)),
      ),
      out_shape=(
          jax.ShapeDtypeStruct((NPAD, F), _f32),
          jax.ShapeDtypeStruct((NPAD, 1), _f32),
      ),
  )(degp, xw1)


def _wphys_body(efa_ref, efb_ref, b1_ref, w2_ref, b2_ref, o_ref):
  hidden = jnp.maximum(efa_ref[...] + efb_ref[...] + b1_ref[...], 0.0)
  sc = jnp.sum(hidden * w2_ref[...], axis=1, keepdims=True) + b2_ref[...]
  o_ref[...] = 1.0 / (1.0 + jnp.exp(-sc))


def _tc_wphys(efa, efb, mlp_b1, mlp_w2, mlp_b2):
  bm = 4000
  grid = (E // bm,)
  return pl.pallas_call(
      _wphys_body,
      grid=grid,
      in_specs=[
          pl.BlockSpec((bm, 32), lambda i: (i, 0)),
          pl.BlockSpec((bm, 32), lambda i: (i, 0)),
          pl.BlockSpec((1, 32), lambda i: (0, 0)),
          pl.BlockSpec((1, 32), lambda i: (0, 0)),
          pl.BlockSpec((1, 1), lambda i: (0, 0)),
      ],
      out_specs=pl.BlockSpec((bm, 1), lambda i: (i, 0)),
      out_shape=jax.ShapeDtypeStruct((E, 1), _f32),
  )(efa, efb, mlp_b1[None, :], mlp_w2.reshape(1, 32), mlp_b2.reshape(1, 1))


def _tc3_body(accp_ref, y1_ref, dinv_ref, b1_ref, w2_ref, y2_ref):
  pre = (accp_ref[0] + accp_ref[1] + y1_ref[...]) * dinv_ref[...] + b1_ref[...]
  h = jnp.where(pre > 0, pre, jnp.exp(jnp.minimum(pre, 0.0)) - 1.0)
  y2_ref[...] = jnp.dot(h, w2_ref[...],
                        preferred_element_type=_f32) * dinv_ref[...]


def _tc3(accp, y1, dinv, b1, w2):
  return pl.pallas_call(
      _tc3_body,
      grid=(NBM,),
      in_specs=[
          pl.BlockSpec((NC, BM, F), lambda i: (0, i, 0)),
          pl.BlockSpec((BM, F), lambda i: (i, 0)),
          pl.BlockSpec((BM, 1), lambda i: (i, 0)),
          pl.BlockSpec((1, F), lambda i: (0, 0)),
          pl.BlockSpec((F, K), lambda i: (0, 0)),
      ],
      out_specs=pl.BlockSpec((BM, K), lambda i: (i, 0)),
      out_shape=jax.ShapeDtypeStruct((NPAD, K), _f32),
  )(accp, y1, dinv, b1[None, :], w2)


def _tc4_body(accp_ref, y2_ref, dinv_ref, b2_ref, z_ref):
  zl = (accp_ref[0] + accp_ref[1] + y2_ref[...]) * dinv_ref[...] + b2_ref[...]
  zmax = jnp.max(zl, axis=1, keepdims=True)
  ez = jnp.exp(zl - zmax)
  z_ref[...] = ez / jnp.sum(ez, axis=1, keepdims=True)


def _tc4(accp, y2, dinv, b2):
  return pl.pallas_call(
      _tc4_body,
      grid=(NBM,),
      in_specs=[
          pl.BlockSpec((NC, BM, K), lambda i: (0, i, 0)),
          pl.BlockSpec((BM, K), lambda i: (i, 0)),
          pl.BlockSpec((BM, 1), lambda i: (i, 0)),
          pl.BlockSpec((1, K), lambda i: (0, 0)),
      ],
      out_specs=pl.BlockSpec((BM, K), lambda i: (i, 0)),
      out_shape=jax.ShapeDtypeStruct((NPAD, K), _f32),
  )(accp, y2, dinv, b2[None, :])


def _tc5_body(z_ref, accp_ref, m_ref, o_ref):
  zsum = z_ref[...] + accp_ref[0] + accp_ref[1]
  o_ref[...] = jnp.dot(zsum, jnp.maximum(m_ref[...], 0.0),
                       preferred_element_type=_f32)


def _tc5(z, accp, m):
  return pl.pallas_call(
      _tc5_body,
      grid=(NBM,),
      in_specs=[
          pl.BlockSpec((BM, K), lambda i: (i, 0)),
          pl.BlockSpec((NC, BM, K), lambda i: (0, i, 0)),
          pl.BlockSpec((K, F), lambda i: (0, 0)),
      ],
      out_specs=pl.BlockSpec((BM, F), lambda i: (i, 0)),
      out_shape=jax.ShapeDtypeStruct((NPAD, F), _f32),
  )(z, accp, m)


def kernel(X, ei_feat, ei_spatial, W1, b1, W2, b2, M, mlp_w1, mlp_b1,
           mlp_w2, mlp_b2):
  Xp = jnp.pad(X, ((0, NPAD - N), (0, 0)))
  dst3 = ei_feat[1].reshape(NW, NCHUNK, CH)
  row3 = ei_spatial[0].reshape(NW, NCHUNK, CH)
  col3 = ei_spatial[1].reshape(NW, NCHUNK, CH)

  wcat = jnp.concatenate([W1, mlp_w1[:F], mlp_w1[F:]], axis=1)
  xwcat = _tc_mm(Xp, wcat)
  xw1 = xwcat[:, :F]
  atab = xwcat[:, F:F + 32]
  btab = xwcat[:, F + 32:]

  degp, efa, efb = _sc_prep(dst3, row3, col3, atab, btab)
  y1, dinv = _tc2(degp, xw1)
  wphys = _tc_wphys(efa, efb, mlp_b1, mlp_w2, mlp_b2)

  acc1p = _sc_segsum(ei_feat[0], ei_feat[1], y1, F, CHS)
  y2 = _tc3(acc1p, y1, dinv, b1, W2)
  acc2p = _sc_segsum(ei_feat[0], ei_feat[1], y2, K, CH)
  z = _tc4(acc2p, y2, dinv, b2)

  acczp = _sc_wseg(row3, col3, wphys.reshape(NW, NCHUNK, CH), z)
  xhat = _tc5(z, acczp, M)

  return z[:N], xhat[:N], wphys


# speedup vs baseline: 1.1913x; 1.1913x over previous
import jax
import jax.numpy as jnp
from jax import lax
from jax.experimental import pallas as pl
from jax.experimental.pallas import tpu as pltpu
from jax.experimental.pallas import tpu_sc as plsc

N = 10000
NPAD = 10240
E = 320000
F = 128
K = 64

NC = 2
NS = 16
NW = NC * NS
EPW = E // NW
CH = 80
NCHUNK = EPW // CH
NB = 5
NT = NCHUNK // NB
CHS = 40
NCHUNKS = EPW // CHS
NTS = NCHUNKS // NB
RPT = NPAD // NS
DEGW = 8

_f32 = jnp.float32
_SC_PARAMS = dict(use_tc_tiling_on_sc=False, needs_layout_passes=False)


def _mesh():
  return plsc.VectorSubcoreMesh(core_axis_name="c", subcore_axis_name="s")


def _ids():
  c = lax.axis_index("c")
  s = lax.axis_index("s")
  return c, s, c * NS + s


def _sc_prep_body(dst3, row3, col3, a_hbm, b_hbm, v_hbm, ones_hbm, zer_hbm,
                  deg_hbm, w_hbm,
                  idd_v, ira_v, icb_v, ra_v, rb_v, v_v, ones_v, wslab_v,
                  deg_s, *sems):
  dsem = sems[0:NB]
  ga = sems[NB:2 * NB]
  gb = sems[2 * NB:3 * NB]
  c, s, w = _ids()
  pltpu.sync_copy(dst3.at[w], idd_v)
  pltpu.sync_copy(row3.at[w], ira_v)
  pltpu.sync_copy(col3.at[w], icb_v)
  pltpu.sync_copy(v_hbm, v_v)
  pltpu.sync_copy(ones_hbm, ones_v)
  pltpu.sync_copy(zer_hbm, deg_s.at[pl.ds(s * RPT, RPT)])
  for b in range(NB):
    pltpu.async_copy(a_hbm.at[ira_v.at[b]], ra_v.at[b], ga[b])
    pltpu.async_copy(b_hbm.at[icb_v.at[b]], rb_v.at[b], gb[b])
  plsc.subcore_barrier()

  lane = jnp.arange(16, dtype=jnp.int32)
  b2v = plsc.load_gather(v_v, [jnp.full((16,), 32, jnp.int32)])

  def rnd(t, carry):
    for b in range(NB):
      g = t * NB + b

      @pl.when(t > 0)
      def _wait_deg():
        pltpu.make_async_copy(ones_v, deg_s.at[idd_v.at[g - NB]],
                              dsem[b]).wait()

      pltpu.async_copy(ones_v, deg_s.at[idd_v.at[g]], dsem[b], add=True)

      pltpu.make_async_copy(a_hbm.at[ira_v.at[g]], ra_v.at[b], ga[b]).wait()
      pltpu.make_async_copy(b_hbm.at[icb_v.at[g]], rb_v.at[b], gb[b]).wait()

      def egrp(eb, ecarry):
        e16 = eb * 16 + lane
        acc = jnp.zeros((16,), _f32)
        for k in range(32):
          k16 = jnp.full((16,), k, jnp.int32)
          av = plsc.load_gather(ra_v.at[b], [e16, k16])
          bv = plsc.load_gather(rb_v.at[b], [e16, k16])
          vk = plsc.load_gather(v_v, [k16])
          acc = acc + jnp.maximum(av + bv, 0.0) * vk
        sgm = 1.0 / (1.0 + jnp.exp(-(acc + b2v)))
        wslab_v[g, pl.ds(eb * 16, 16)] = sgm
        return ecarry

      lax.fori_loop(0, CH // 16, egrp, 0)

    @pl.when(t < NT - 1)
    def _next():
      for b in range(NB):
        g2 = t * NB + b + NB
        pltpu.async_copy(a_hbm.at[ira_v.at[g2]], ra_v.at[b], ga[b])
        pltpu.async_copy(b_hbm.at[icb_v.at[g2]], rb_v.at[b], gb[b])

    return carry

  lax.fori_loop(0, NT, rnd, 0)
  for b in range(NB):
    g = (NT - 1) * NB + b
    pltpu.make_async_copy(ones_v, deg_s.at[idd_v.at[g]], dsem[b]).wait()
  pltpu.sync_copy(wslab_v, w_hbm.at[w])
  plsc.subcore_barrier()
  pltpu.sync_copy(deg_s.at[pl.ds(s * RPT, RPT)],
                  deg_hbm.at[c, pl.ds(s * RPT, RPT)])


def _sc_prep(dst3, row3, col3, atab, btab, vpad):
  fn = pl.kernel(
      _sc_prep_body,
      out_type=(
          jax.ShapeDtypeStruct((NC, NPAD, DEGW), _f32),
          jax.ShapeDtypeStruct((NW, NCHUNK, CH), _f32),
      ),
      mesh=_mesh(),
      compiler_params=pltpu.CompilerParams(**_SC_PARAMS),
      scratch_types=[
          pltpu.VMEM((NCHUNK, CH), jnp.int32),
          pltpu.VMEM((NCHUNK, CH), jnp.int32),
          pltpu.VMEM((NCHUNK, CH), jnp.int32),
          pltpu.VMEM((NB, CH, 32), _f32),
          pltpu.VMEM((NB, CH, 32), _f32),
          pltpu.VMEM((40,), _f32),
          pltpu.VMEM((CH, DEGW), _f32),
          pltpu.VMEM((NCHUNK, CH), _f32),
          pltpu.VMEM_SHARED((NPAD, DEGW), _f32),
      ] + [pltpu.SemaphoreType.DMA] * (3 * NB),
  )
  ones = jnp.ones((CH, DEGW), _f32)
  zer = jnp.zeros((RPT, DEGW), _f32)
  return fn(dst3, row3, col3, atab, btab, vpad, ones, zer)


def _sc_seg_body(src3, dst3, y_hbm, zer_hbm, acc_hbm,
                 isrc_v, idst_v, rows_v, acc_s, *sems, ch, nt):
  gy = sems[0:NB]
  ss = sems[NB:2 * NB]
  c, s, w = _ids()
  pltpu.sync_copy(src3.at[w], isrc_v)
  pltpu.sync_copy(dst3.at[w], idst_v)
  pltpu.sync_copy(zer_hbm, acc_s.at[pl.ds(s * RPT, RPT)])
  for b in range(NB):
    pltpu.async_copy(y_hbm.at[isrc_v.at[b]], rows_v.at[b], gy[b])
  plsc.subcore_barrier()

  def rnd(t, carry):
    for b in range(NB):
      g = t * NB + b
      pltpu.make_async_copy(y_hbm.at[isrc_v.at[g]], rows_v.at[b], gy[b]).wait()
      pltpu.async_copy(rows_v.at[b], acc_s.at[idst_v.at[g]], ss[b], add=True)

    @pl.when(t < nt - 1)
    def _next():
      for b in range(NB):
        g = t * NB + b
        pltpu.make_async_copy(rows_v.at[b], acc_s.at[idst_v.at[g]],
                              ss[b]).wait()
        pltpu.async_copy(y_hbm.at[isrc_v.at[g + NB]], rows_v.at[b], gy[b])

    return carry

  lax.fori_loop(0, nt, rnd, 0)
  for b in range(NB):
    g = (nt - 1) * NB + b
    pltpu.make_async_copy(rows_v.at[b], acc_s.at[idst_v.at[g]], ss[b]).wait()
  plsc.subcore_barrier()
  pltpu.sync_copy(acc_s.at[pl.ds(s * RPT, RPT)],
                  acc_hbm.at[c, pl.ds(s * RPT, RPT)])


def _sc_segsum(src, dst, y, d, ch):
  import functools as _ft
  nchunk = EPW // ch
  nt = nchunk // NB
  body = _ft.partial(_sc_seg_body, ch=ch, nt=nt)
  fn = pl.kernel(
      body,
      out_type=jax.ShapeDtypeStruct((NC, NPAD, d), _f32),
      mesh=_mesh(),
      compiler_params=pltpu.CompilerParams(**_SC_PARAMS),
      scratch_types=[
          pltpu.VMEM((nchunk, ch), jnp.int32),
          pltpu.VMEM((nchunk, ch), jnp.int32),
          pltpu.VMEM((NB, ch, d), _f32),
          pltpu.VMEM_SHARED((NPAD, d), _f32),
      ] + [pltpu.SemaphoreType.DMA] * (2 * NB),
  )
  return fn(src.reshape(NW, nchunk, ch), dst.reshape(NW, nchunk, ch), y,
            jnp.zeros((RPT, d), _f32))


def _sc_wseg_body(row3, col3, w3, z_hbm, zer_hbm, acc_hbm,
                  irow_v, icol_v, wv_v, rows_v, acc_s, *sems):
  gz = sems[0:NB]
  ss = sems[NB:2 * NB]
  c, s, w = _ids()
  pltpu.sync_copy(row3.at[w], irow_v)
  pltpu.sync_copy(col3.at[w], icol_v)
  pltpu.sync_copy(w3.at[w], wv_v)
  pltpu.sync_copy(zer_hbm, acc_s.at[pl.ds(s * RPT, RPT)])
  for b in range(NB):
    pltpu.async_copy(z_hbm.at[icol_v.at[b]], rows_v.at[b], gz[b])
  plsc.subcore_barrier()

  def rnd(t, carry):
    for b in range(NB):
      g = t * NB + b
      pltpu.make_async_copy(z_hbm.at[icol_v.at[g]], rows_v.at[b], gz[b]).wait()

      def edge(e, ecarry):
        wb = plsc.load_gather(
            wv_v,
            [jnp.full((16,), g, jnp.int32), jnp.full((16,), e, jnp.int32)])
        for j in range(K // 16):
          rows_v[b, e, pl.ds(j * 16, 16)] = (
              rows_v[b, e, pl.ds(j * 16, 16)] * wb)
        return ecarry

      lax.fori_loop(0, CH, edge, 0)
      pltpu.async_copy(rows_v.at[b], acc_s.at[irow_v.at[g]], ss[b], add=True)

    @pl.when(t < NT - 1)
    def _next():
      for b in range(NB):
        g = t * NB + b
        pltpu.make_async_copy(rows_v.at[b], acc_s.at[irow_v.at[g]],
                              ss[b]).wait()
        pltpu.async_copy(z_hbm.at[icol_v.at[g + NB]], rows_v.at[b], gz[b])

    return carry

  lax.fori_loop(0, NT, rnd, 0)
  for b in range(NB):
    g = (NT - 1) * NB + b
    pltpu.make_async_copy(rows_v.at[b], acc_s.at[irow_v.at[g]], ss[b]).wait()
  plsc.subcore_barrier()
  pltpu.sync_copy(acc_s.at[pl.ds(s * RPT, RPT)],
                  acc_hbm.at[c, pl.ds(s * RPT, RPT)])


def _sc_wseg(row3, col3, w3, z):
  fn = pl.kernel(
      _sc_wseg_body,
      out_type=jax.ShapeDtypeStruct((NC, NPAD, K), _f32),
      mesh=_mesh(),
      compiler_params=pltpu.CompilerParams(**_SC_PARAMS),
      scratch_types=[
          pltpu.VMEM((NCHUNK, CH), jnp.int32),
          pltpu.VMEM((NCHUNK, CH), jnp.int32),
          pltpu.VMEM((NCHUNK, CH), _f32),
          pltpu.VMEM((NB, CH, K), _f32),
          pltpu.VMEM_SHARED((NPAD, K), _f32),
      ] + [pltpu.SemaphoreType.DMA] * (2 * NB),
  )
  return fn(row3, col3, w3, z, jnp.zeros((RPT, K), _f32))


BM = 1024
NBM = NPAD // BM


def _mm_body(x_ref, w_ref, b_ref, o_ref):
  o_ref[...] = jnp.dot(x_ref[...], w_ref[...],
                       preferred_element_type=_f32) + b_ref[...]


def _tc_mm(x, w, brow):
  kn = w.shape[1]
  return pl.pallas_call(
      _mm_body,
      grid=(NBM,),
      in_specs=[
          pl.BlockSpec((BM, F), lambda i: (i, 0)),
          pl.BlockSpec((F, kn), lambda i: (0, 0)),
          pl.BlockSpec((1, kn), lambda i: (0, 0)),
      ],
      out_specs=pl.BlockSpec((BM, kn), lambda i: (i, 0)),
      out_shape=jax.ShapeDtypeStruct((NPAD, kn), _f32),
  )(x, w, brow)


def _tc2_body(degp_ref, xw_ref, y1_ref, dinv_ref):
  deg = degp_ref[0] + degp_ref[1]
  dinv = lax.rsqrt(deg[:, 0:1] + 1.0)
  dinv_ref[...] = dinv
  y1_ref[...] = xw_ref[...] * dinv


def _tc2(degp, xw1):
  return pl.pallas_call(
      _tc2_body,
      grid=(NBM,),
      in_specs=[
          pl.BlockSpec((NC, BM, DEGW), lambda i: (0, i, 0)),
          pl.BlockSpec((BM, F), lambda i: (i, 0)),
      ],
      out_specs=(
          pl.BlockSpec((BM, F), lambda i: (i, 0)),
          pl.BlockSpec((BM, 1), lambda i: (i, 0)),
      ),
      out_shape=(
          jax.ShapeDtypeStruct((NPAD, F), _f32),
          jax.ShapeDtypeStruct((NPAD, 1), _f32),
      ),
  )(degp, xw1)


def _tc3_body(accp_ref, y1_ref, dinv_ref, b1_ref, w2_ref, y2_ref):
  pre = (accp_ref[0] + accp_ref[1] + y1_ref[...]) * dinv_ref[...] + b1_ref[...]
  h = jnp.where(pre > 0, pre, jnp.exp(jnp.minimum(pre, 0.0)) - 1.0)
  y2_ref[...] = jnp.dot(h, w2_ref[...],
                        preferred_element_type=_f32) * dinv_ref[...]


def _tc3(accp, y1, dinv, b1, w2):
  return pl.pallas_call(
      _tc3_body,
      grid=(NBM,),
      in_specs=[
          pl.BlockSpec((NC, BM, F), lambda i: (0, i, 0)),
          pl.BlockSpec((BM, F), lambda i: (i, 0)),
          pl.BlockSpec((BM, 1), lambda i: (i, 0)),
          pl.BlockSpec((1, F), lambda i: (0, 0)),
          pl.BlockSpec((F, K), lambda i: (0, 0)),
      ],
      out_specs=pl.BlockSpec((BM, K), lambda i: (i, 0)),
      out_shape=jax.ShapeDtypeStruct((NPAD, K), _f32),
  )(accp, y1, dinv, b1[None, :], w2)


def _tc4_body(accp_ref, y2_ref, dinv_ref, b2_ref, z_ref):
  zl = (accp_ref[0] + accp_ref[1] + y2_ref[...]) * dinv_ref[...] + b2_ref[...]
  zmax = jnp.max(zl, axis=1, keepdims=True)
  ez = jnp.exp(zl - zmax)
  z_ref[...] = ez / jnp.sum(ez, axis=1, keepdims=True)


def _tc4(accp, y2, dinv, b2):
  return pl.pallas_call(
      _tc4_body,
      grid=(NBM,),
      in_specs=[
          pl.BlockSpec((NC, BM, K), lambda i: (0, i, 0)),
          pl.BlockSpec((BM, K), lambda i: (i, 0)),
          pl.BlockSpec((BM, 1), lambda i: (i, 0)),
          pl.BlockSpec((1, K), lambda i: (0, 0)),
      ],
      out_specs=pl.BlockSpec((BM, K), lambda i: (i, 0)),
      out_shape=jax.ShapeDtypeStruct((NPAD, K), _f32),
  )(accp, y2, dinv, b2[None, :])


def _tc5_body(z_ref, accp_ref, m_ref, o_ref):
  zsum = z_ref[...] + accp_ref[0] + accp_ref[1]
  o_ref[...] = jnp.dot(zsum, jnp.maximum(m_ref[...], 0.0),
                       preferred_element_type=_f32)


def _tc5(z, accp, m):
  return pl.pallas_call(
      _tc5_body,
      grid=(NBM,),
      in_specs=[
          pl.BlockSpec((BM, K), lambda i: (i, 0)),
          pl.BlockSpec((NC, BM, K), lambda i: (0, i, 0)),
          pl.BlockSpec((K, F), lambda i: (0, 0)),
      ],
      out_specs=pl.BlockSpec((BM, F), lambda i: (i, 0)),
      out_shape=jax.ShapeDtypeStruct((NPAD, F), _f32),
  )(z, accp, m)


def kernel(X, ei_feat, ei_spatial, W1, b1, W2, b2, M, mlp_w1, mlp_b1,
           mlp_w2, mlp_b2):
  Xp = jnp.pad(X, ((0, NPAD - N), (0, 0)))
  dst3 = ei_feat[1].reshape(NW, NCHUNK, CH)
  row3 = ei_spatial[0].reshape(NW, NCHUNK, CH)
  col3 = ei_spatial[1].reshape(NW, NCHUNK, CH)

  wcat = jnp.concatenate([W1, mlp_w1[:F], mlp_w1[F:]], axis=1)
  bcat = jnp.concatenate([jnp.zeros((F,), _f32), mlp_b1,
                          jnp.zeros((32,), _f32)])[None, :]
  vpad = jnp.concatenate([mlp_w2[:, 0], mlp_b2,
                          jnp.zeros((7,), _f32)])
  xwcat = _tc_mm(Xp, wcat, bcat)
  xw1 = xwcat[:, :F]
  atab = xwcat[:, F:F + 32]
  btab = xwcat[:, F + 32:]

  degp, wslab = _sc_prep(dst3, row3, col3, atab, btab, vpad)
  y1, dinv = _tc2(degp, xw1)

  acc1p = _sc_segsum(ei_feat[0], ei_feat[1], y1, F, CHS)
  y2 = _tc3(acc1p, y1, dinv, b1, W2)
  acc2p = _sc_segsum(ei_feat[0], ei_feat[1], y2, K, CH)
  z = _tc4(acc2p, y2, dinv, b2)

  acczp = _sc_wseg(row3, col3, wslab, z)
  xhat = _tc5(z, acczp, M)

  return z[:N], xhat[:N], wslab.reshape(E, 1)
